# baseline (speedup 1.0000x reference)
import functools

import jax
import jax.numpy as jnp
from jax import lax
from jax.experimental import pallas as pl
from jax.experimental.pallas import tpu as pltpu
from jax.experimental.pallas import tpu_sc as plsc

VOCAB = 1000000
D = 32
H = 50
B = 16384
NW = 32
CB = 1024
GW = 128
N_UNITS = H * (B // CB)
UNITS_PER_W = N_UNITS // NW
BW = B // NW

_MESH = plsc.VectorSubcoreMesh(core_axis_name="c", subcore_axis_name="s")
_SC_PARAMS = pltpu.CompilerParams(
    use_tc_tiling_on_sc=False, needs_layout_passes=False
)


def _flatten_idx(textT):
    @functools.partial(
        pl.kernel,
        out_type=jax.ShapeDtypeStruct((H * B,), jnp.int32),
        mesh=_MESH,
        scratch_types=[pltpu.VMEM((8, BW), jnp.int32)],
        compiler_params=pltpu.CompilerParams(
            use_tc_tiling_on_sc=True, needs_layout_passes=False
        ),
    )
    def k0(textT_hbm, out_hbm, buf):
        wid = lax.axis_index("s") * 2 + lax.axis_index("c")
        b0 = wid * BW
        for band in range(H // 8 + 1):
            nh = min(8, H - band * 8)
            pltpu.sync_copy(
                textT_hbm.at[pl.ds(band * 8, nh), pl.ds(b0, BW)],
                buf.at[pl.ds(0, nh)],
            )
            for hl in range(nh):
                h = band * 8 + hl
                pltpu.sync_copy(
                    buf.at[hl], out_hbm.at[pl.ds(h * B + b0, BW)]
                )

    return k0(textT)


RC = 512
RP = D + 1
CHUNKS_W = VOCAB // RC // NW
T_BASE = CHUNKS_W * NW * RC


def _table_rm(tableT, tail):
    @functools.partial(
        pl.kernel,
        out_type=jax.ShapeDtypeStruct((VOCAB * RP,), jnp.float32),
        mesh=_MESH,
        scratch_types=[
            pltpu.VMEM((D, RC), jnp.float32),
            pltpu.VMEM((D, RC), jnp.float32),
            pltpu.VMEM((RC * RP,), jnp.float32),
            pltpu.VMEM((RC * RP,), jnp.float32),
            pltpu.SemaphoreType.DMA,
            pltpu.SemaphoreType.DMA,
            pltpu.SemaphoreType.DMA,
        ],
        compiler_params=pltpu.CompilerParams(
            use_tc_tiling_on_sc=True, needs_layout_passes=False
        ),
    )
    def kt(tT_hbm, tail_hbm, out_hbm, bands0, bands1, rows0, rows1,
           rs0, rs1, ws):
        wid = lax.axis_index("s") * 2 + lax.axis_index("c")
        iota = lax.iota(jnp.int32, 16)
        bands = (bands0, bands1)
        rows = (rows0, rows1)
        rsems = (rs0, rs1)

        def fire_read(j, rb=None, nr=RC):
            s = j % 2
            if rb is None:
                rb = (j * NW + wid) * RC
            return [
                pltpu.async_copy(
                    tT_hbm.at[pl.ds(cb * 8, 8), pl.ds(rb, nr)],
                    bands[s].at[pl.ds(cb * 8, 8), pl.ds(0, nr)],
                    rsems[s],
                )
                for cb in range(D // 8)
            ]

        def transpose(s, nr=RC):
            riota = iota * RP

            @plsc.parallel_loop(0, nr // 16, 1, unroll=1)
            def body(r16):
                r0 = r16 * 16

                def cbody(ch, carry):
                    for cl in range(16):
                        c = ch * 16 + cl
                        v = bands[s][c, pl.ds(r0, 16)]
                        plsc.store_scatter(
                            rows[s], [riota + (r0 * RP + c)], v
                        )
                    return carry

                lax.fori_loop(0, D // 16, cbody, 0)

        gh = fire_read(0)
        wh = []
        for j in range(CHUNKS_W):
            gh_next = fire_read(j + 1) if j + 1 < CHUNKS_W else []
            for hd in gh:
                hd.wait()
            for hd in wh:
                hd.wait()
            transpose(j % 2)
            wh = [
                pltpu.async_copy(
                    rows[j % 2],
                    out_hbm.at[pl.ds((j * NW + wid) * RC * RP, RC * RP)],
                    ws,
                )
            ]
            gh = gh_next
        for hd in wh:
            hd.wait()

        @pl.when(wid == 0)
        def _():
            for hd in fire_read(0, rb=T_BASE, nr=RC):
                hd.wait()
            transpose(0)
            pltpu.sync_copy(
                rows0, out_hbm.at[pl.ds(T_BASE * RP, RC * RP)]
            )

        @pl.when(wid == 1)
        def _():
            nr = VOCAB - T_BASE - RC
            pltpu.sync_copy(tail_hbm, rows0.at[pl.ds(0, nr * D)])

            @plsc.parallel_loop(0, nr, 1, unroll=2)
            def body(r):
                for ch in range(2):
                    v = rows0[pl.ds(r * D + ch * 16, 16)]
                    plsc.store_scatter(
                        rows1, [iota + (r * RP + ch * 16)], v
                    )

            pltpu.sync_copy(
                rows1.at[pl.ds(0, nr * RP)],
                out_hbm.at[pl.ds((T_BASE + RC) * RP, nr * RP)],
            )

    return kt(tableT, tail)


def _emb_lookup(table, idx_hm):
    @functools.partial(
        pl.kernel,
        out_type=jax.ShapeDtypeStruct((H, D, B), jnp.float32),
        mesh=_MESH,
        scratch_types=[
            pltpu.VMEM((CB,), jnp.int32),
            pltpu.VMEM((CB,), jnp.int32),
            pltpu.VMEM((CB, RP), jnp.float32),
            pltpu.VMEM((CB, RP), jnp.float32),
            pltpu.VMEM((D, CB + 17), jnp.float32),
            pltpu.SemaphoreType.DMA,
            pltpu.SemaphoreType.DMA,
            pltpu.SemaphoreType.DMA,
        ],
        compiler_params=_SC_PARAMS,
    )
    def k(table_hbm, idx_hbm, out_hbm, idx0, idx1, rows0, rows1, obuf,
          gsem0, gsem1, wsem):
        wid = lax.axis_index("s") * 2 + lax.axis_index("c")
        iota = lax.iota(jnp.int32, 16)
        cvec0 = iota
        cvec1 = iota + 16

        idx_bufs = (idx0, idx1)
        row_bufs = (rows0, rows1)
        gsems = (gsem0, gsem1)

        def unit_hb(u):
            unit = u * NW + wid
            return unit // (B // CB), unit % (B // CB)

        def fire(u):
            h, bt8 = unit_hb(u)
            s = u % 2
            pltpu.sync_copy(
                idx_hbm.at[pl.ds(h * B + bt8 * CB, CB)], idx_bufs[s]
            )
            return [
                pltpu.async_copy(
                    table_hbm.at[idx_bufs[s].at[pl.ds(j * GW, GW)]],
                    row_bufs[s].at[pl.ds(j * GW, GW)],
                    gsems[s],
                )
                for j in range(CB // GW)
            ]

        def transpose(rows):
            @plsc.parallel_loop(0, CB, 1, unroll=8)
            def body(i):
                iv = jnp.full((16,), 0, jnp.int32) + i
                v0 = rows[i, pl.ds(0, 16)]
                v1 = rows[i, pl.ds(16, 16)]
                plsc.store_scatter(obuf, [cvec0, iv], v0)
                plsc.store_scatter(obuf, [cvec1, iv], v1)

        gh = fire(0)
        wh = []
        for u in range(UNITS_PER_W):
            if u + 1 < UNITS_PER_W:
                gh_next = fire(u + 1)
            else:
                gh_next = []
            for hd in gh:
                hd.wait()
            for hd in wh:
                hd.wait()
            transpose(row_bufs[u % 2])
            h, bt8 = unit_hb(u)
            wh = [
                pltpu.async_copy(
                    obuf.at[:, pl.ds(0, CB)],
                    out_hbm.at[h, :, pl.ds(bt8 * CB, CB)],
                    wsem,
                )
            ]
            gh = gh_next
        for hd in wh:
            hd.wait()

    return k(table, idx_hm)


def kernel(text, table):
    textT = text.T.astype(jnp.int32)
    idx_hm = _flatten_idx(textT)
    tail = table[T_BASE + RC:].reshape(-1)
    tbl_rm = _table_rm(table.T, tail).reshape(VOCAB, RP)
    out_hcb = _emb_lookup(tbl_rm, idx_hm)
    return out_hcb.transpose(2, 0, 1)


# speedup vs baseline: 2.7581x; 2.7581x over previous
import functools

import jax
import jax.numpy as jnp
from jax import lax
from jax.experimental import pallas as pl
from jax.experimental.pallas import tpu as pltpu
from jax.experimental.pallas import tpu_sc as plsc

VOCAB = 1000000
D = 32
H = 50
B = 16384
NW = 32
CB = 1024
GW = 128
N_UNITS = H * (B // CB)
UNITS_PER_W = N_UNITS // NW
BW = B // NW

_MESH = plsc.VectorSubcoreMesh(core_axis_name="c", subcore_axis_name="s")
_SC_PARAMS = pltpu.CompilerParams(
    use_tc_tiling_on_sc=False, needs_layout_passes=False
)


def _flatten_idx(textT):
    @functools.partial(
        pl.kernel,
        out_type=jax.ShapeDtypeStruct((H * B,), jnp.int32),
        mesh=_MESH,
        scratch_types=[pltpu.VMEM((8, BW), jnp.int32)],
        compiler_params=pltpu.CompilerParams(
            use_tc_tiling_on_sc=True, needs_layout_passes=False
        ),
    )
    def k0(textT_hbm, out_hbm, buf):
        wid = lax.axis_index("s") * 2 + lax.axis_index("c")
        b0 = wid * BW
        for band in range(H // 8 + 1):
            nh = min(8, H - band * 8)
            pltpu.sync_copy(
                textT_hbm.at[pl.ds(band * 8, nh), pl.ds(b0, BW)],
                buf.at[pl.ds(0, nh)],
            )
            for hl in range(nh):
                h = band * 8 + hl
                pltpu.sync_copy(
                    buf.at[hl], out_hbm.at[pl.ds(h * B + b0, BW)]
                )

    return k0(textT)


def _emb_lookup(table, idx_hm):
    @functools.partial(
        pl.kernel,
        out_type=jax.ShapeDtypeStruct((H, D, B), jnp.float32),
        mesh=_MESH,
        scratch_types=[
            pltpu.VMEM((CB,), jnp.int32),
            pltpu.VMEM((CB,), jnp.int32),
            pltpu.VMEM((CB, D), jnp.float32),
            pltpu.VMEM((CB, D), jnp.float32),
            pltpu.VMEM((D, CB + 17), jnp.float32),
            pltpu.SemaphoreType.DMA,
            pltpu.SemaphoreType.DMA,
            pltpu.SemaphoreType.DMA,
        ],
        compiler_params=_SC_PARAMS,
    )
    def k(table_hbm, idx_hbm, out_hbm, idx0, idx1, rows0, rows1, obuf,
          gsem0, gsem1, wsem):
        wid = lax.axis_index("s") * 2 + lax.axis_index("c")
        iota = lax.iota(jnp.int32, 16)
        cvec0 = iota
        cvec1 = iota + 16

        idx_bufs = (idx0, idx1)
        row_bufs = (rows0, rows1)
        gsems = (gsem0, gsem1)

        def unit_hb(u):
            unit = u * NW + wid
            return unit // (B // CB), unit % (B // CB)

        def fire(u):
            h, bt8 = unit_hb(u)
            s = u % 2
            pltpu.sync_copy(
                idx_hbm.at[pl.ds(h * B + bt8 * CB, CB)], idx_bufs[s]
            )
            return [
                pltpu.async_copy(
                    table_hbm.at[idx_bufs[s].at[pl.ds(j * GW, GW)]],
                    row_bufs[s].at[pl.ds(j * GW, GW)],
                    gsems[s],
                )
                for j in range(CB // GW)
            ]

        def transpose(rows):
            @plsc.parallel_loop(0, CB, 1, unroll=8)
            def body(i):
                iv = jnp.full((16,), 0, jnp.int32) + i
                v0 = rows[i, pl.ds(0, 16)]
                v1 = rows[i, pl.ds(16, 16)]
                plsc.store_scatter(obuf, [cvec0, iv], v0)
                plsc.store_scatter(obuf, [cvec1, iv], v1)

        gh = fire(0)
        wh = []
        for u in range(UNITS_PER_W):
            if u + 1 < UNITS_PER_W:
                gh_next = fire(u + 1)
            else:
                gh_next = []
            for hd in gh:
                hd.wait()
            for hd in wh:
                hd.wait()
            transpose(row_bufs[u % 2])
            h, bt8 = unit_hb(u)
            wh = [
                pltpu.async_copy(
                    obuf.at[:, pl.ds(0, CB)],
                    out_hbm.at[h, :, pl.ds(bt8 * CB, CB)],
                    wsem,
                )
            ]
            gh = gh_next
        for hd in wh:
            hd.wait()

    return k(table, idx_hm)


def kernel(text, table):
    textT = text.T.astype(jnp.int32)
    idx_hm = _flatten_idx(textT)
    out_hcb = _emb_lookup(table, idx_hm)
    return out_hcb.transpose(2, 0, 1)


# speedup vs baseline: 3.1957x; 1.1586x over previous
import functools

import jax
import jax.numpy as jnp
from jax import lax
from jax.experimental import pallas as pl
from jax.experimental.pallas import tpu as pltpu
from jax.experimental.pallas import tpu_sc as plsc

VOCAB = 1000000
D = 32
H = 50
B = 16384
NW = 32
CB = 1024
GW = 128
N_UNITS = H * (B // CB)
UNITS_PER_W = N_UNITS // NW
BW = B // NW

_MESH = plsc.VectorSubcoreMesh(core_axis_name="c", subcore_axis_name="s")
_SC_PARAMS = pltpu.CompilerParams(
    use_tc_tiling_on_sc=False, needs_layout_passes=False
)


def _flatten_idx(textT):
    @functools.partial(
        pl.kernel,
        out_type=jax.ShapeDtypeStruct((H * B,), jnp.int32),
        mesh=_MESH,
        scratch_types=[pltpu.VMEM((8, BW), jnp.int32)],
        compiler_params=pltpu.CompilerParams(
            use_tc_tiling_on_sc=True, needs_layout_passes=False
        ),
    )
    def k0(textT_hbm, out_hbm, buf):
        wid = lax.axis_index("s") * 2 + lax.axis_index("c")
        b0 = wid * BW
        for band in range(H // 8 + 1):
            nh = min(8, H - band * 8)
            pltpu.sync_copy(
                textT_hbm.at[pl.ds(band * 8, nh), pl.ds(b0, BW)],
                buf.at[pl.ds(0, nh)],
            )
            for hl in range(nh):
                h = band * 8 + hl
                pltpu.sync_copy(
                    buf.at[hl], out_hbm.at[pl.ds(h * B + b0, BW)]
                )

    return k0(textT)


def _emb_lookup(table, idx_hm):
    @functools.partial(
        pl.kernel,
        out_type=jax.ShapeDtypeStruct((H, 4, B // CB, 8, 8, 128), jnp.float32),
        mesh=_MESH,
        scratch_types=[
            pltpu.VMEM((CB,), jnp.int32),
            pltpu.VMEM((CB,), jnp.int32),
            pltpu.VMEM((CB, D), jnp.float32),
            pltpu.VMEM((CB, D), jnp.float32),
            pltpu.VMEM((D, CB + 17), jnp.float32),
            pltpu.SemaphoreType.DMA,
            pltpu.SemaphoreType.DMA,
            pltpu.SemaphoreType.DMA,
        ],
        compiler_params=_SC_PARAMS,
    )
    def k(table_hbm, idx_hbm, out_hbm, idx0, idx1, rows0, rows1, obuf,
          gsem0, gsem1, wsem):
        wid = lax.axis_index("s") * 2 + lax.axis_index("c")
        iota = lax.iota(jnp.int32, 16)
        cvec0 = iota
        cvec1 = iota + 16

        idx_bufs = (idx0, idx1)
        row_bufs = (rows0, rows1)
        gsems = (gsem0, gsem1)

        def unit_hb(u):
            unit = u * NW + wid
            return unit // (B // CB), unit % (B // CB)

        def fire(u):
            h, bt8 = unit_hb(u)
            s = u % 2
            pltpu.sync_copy(
                idx_hbm.at[pl.ds(h * B + bt8 * CB, CB)], idx_bufs[s]
            )
            return [
                pltpu.async_copy(
                    table_hbm.at[idx_bufs[s].at[pl.ds(j * GW, GW)]],
                    row_bufs[s].at[pl.ds(j * GW, GW)],
                    gsems[s],
                )
                for j in range(CB // GW)
            ]

        def transpose(rows):
            @plsc.parallel_loop(0, CB, 1, unroll=8)
            def body(i):
                iv = jnp.full((16,), 0, jnp.int32) + i
                v0 = rows[i, pl.ds(0, 16)]
                v1 = rows[i, pl.ds(16, 16)]
                plsc.store_scatter(obuf, [cvec0, iv], v0)
                plsc.store_scatter(obuf, [cvec1, iv], v1)

        def fire_writes(h, bt8):
            def wfire(cbl, carry):
                cb = cbl // 8
                btl = cbl % 8
                pltpu.async_copy(
                    obuf.at[pl.ds(cb * 8, 8), pl.ds(btl * 128, 128)],
                    out_hbm.at[h, cb, bt8, btl],
                    wsem,
                )
                return carry

            lax.fori_loop(0, 32, wfire, 0)

        def drain_writes():
            def wdrain(i, carry):
                pltpu.make_async_copy(
                    obuf.at[pl.ds(0, 8), pl.ds(0, 128)],
                    out_hbm.at[0, 0, 0, 0],
                    wsem,
                ).wait()
                return carry

            lax.fori_loop(0, 32, wdrain, 0)

        gh = fire(0)
        for u in range(UNITS_PER_W):
            if u + 1 < UNITS_PER_W:
                gh_next = fire(u + 1)
            else:
                gh_next = []
            for hd in gh:
                hd.wait()
            if u > 0:
                drain_writes()
            transpose(row_bufs[u % 2])
            h, bt8 = unit_hb(u)
            fire_writes(h, bt8)
            gh = gh_next
        drain_writes()

    return k(table, idx_hm)


def kernel(text, table):
    textT = text.T.astype(jnp.int32)
    idx_hm = _flatten_idx(textT)
    out5 = _emb_lookup(table, idx_hm)
    return out5.transpose(2, 3, 5, 0, 1, 4).reshape(B, H, D)
